# Initial kernel scaffold; baseline (speedup 1.0000x reference)
import jax
import jax.numpy as jnp
from jax.experimental import pallas as pl


def kernel(x_categorical, x_continuous, emb_table, fc_table, bias, cont_w, cont_b, w1, b1, w2, b2, w3, b3):
    raise NotImplementedError("write your pallas kernel here")


# speedup vs baseline: 1.9364x; 1.9364x over previous
import functools

import jax
import jax.numpy as jnp
from jax import lax
from jax.experimental import pallas as pl
from jax.experimental.pallas import tpu as pltpu
from jax.experimental.pallas import tpu_sc as plsc

_V = 1000000
_B = 16384
_F = 26
_D = 32
_C = 19
_CD = _C * _D
_ED = _F * _D
_NW = 32
_N = _B * _F
_G = 128
_NROWS = _N // _G
_ROWS_PER_W = _NROWS // _NW
_SUB = 8
_STEPS = _ROWS_PER_W // _SUB

_TB = 1024
_CP = 32


@functools.cache
def _sc_gather_fn():
    mesh = plsc.VectorSubcoreMesh(core_axis_name="c", subcore_axis_name="s")

    @functools.partial(
        pl.kernel,
        mesh=mesh,
        compiler_params=pltpu.CompilerParams(use_tc_tiling_on_sc=False,
                                             needs_layout_passes=False),
        out_type=[
            jax.ShapeDtypeStruct((_NROWS, _G, _D), jnp.float32),
            jax.ShapeDtypeStruct((_N,), jnp.float32),
        ],
        scratch_types=[
            pltpu.VMEM((_SUB * _G,), jnp.int32),
            pltpu.VMEM((_SUB * _G,), jnp.int32),
            pltpu.VMEM((_SUB, _G, _D), jnp.float32),
            pltpu.VMEM((_SUB * _G, 16), jnp.float32),
            pltpu.VMEM((_SUB * _G,), jnp.float32),
            pltpu.SemaphoreType.DMA,
            pltpu.SemaphoreType.DMA,
        ],
    )
    def sc_gather(emb_hbm, fc16_hbm, idx_hbm, out_e, out_f,
                  idx_v, ridx_v, rows_v, fcrows_v, fcv_v, sem_e, sem_f):
        wid = lax.axis_index("s") * 2 + lax.axis_index("c")
        base = wid * _ROWS_PER_W
        iota = lax.iota(jnp.int32, 16)
        ck = _SUB * _G

        def step(s, carry):
            row0 = base + s * _SUB
            k0 = row0 * _G
            pltpu.sync_copy(idx_hbm.at[pl.ds(k0, ck)], idx_v)
            emb_cp = [
                pltpu.async_copy(emb_hbm.at[idx_v.at[pl.ds(j * _G, _G)]],
                                 rows_v.at[j], sem_e)
                for j in range(_SUB)
            ]
            def rbody(t, c):
                v = idx_v[pl.ds(t * 16, 16)]
                ridx_v[pl.ds(t * 16, 16)] = jnp.right_shift(v, 4)
                return c

            lax.fori_loop(0, ck // 16, rbody, 0)
            fc_cp = [
                pltpu.async_copy(fc16_hbm.at[ridx_v.at[pl.ds(j * _G, _G)]],
                                 fcrows_v.at[pl.ds(j * _G, _G)], sem_f)
                for j in range(_SUB)
            ]
            for cp in emb_cp:
                cp.wait()
            for cp in fc_cp:
                cp.wait()

            def ebody(t, c):
                v = idx_v[pl.ds(t * 16, 16)]
                col = jnp.bitwise_and(v, 15)
                kvec = t * 16 + iota
                fcv_v[pl.ds(t * 16, 16)] = plsc.load_gather(
                    fcrows_v, [kvec, col])
                return c

            lax.fori_loop(0, ck // 16, ebody, 0)
            pltpu.sync_copy(rows_v, out_e.at[pl.ds(row0, _SUB)])
            pltpu.sync_copy(fcv_v, out_f.at[pl.ds(k0, ck)])
            return carry

        lax.fori_loop(0, _STEPS, step, 0)

    return sc_gather


def _split_dot(a, s_bf16):
    a_hi = a.astype(jnp.bfloat16)
    a_lo = (a - a_hi.astype(jnp.float32)).astype(jnp.bfloat16)
    return (jnp.dot(a_hi, s_bf16, preferred_element_type=jnp.float32)
            + jnp.dot(a_lo, s_bf16, preferred_element_type=jnp.float32))


def _tc_body(ex_ref, xc_ref, fcv_ref, cw_ref, cb_ref, w1e_ref, w1c_ref,
             b1_ref, w2_ref, b2_ref, w3_ref, se_ref, sc_ref, c0_ref,
             out_ref):
    dot = functools.partial(jnp.dot, preferred_element_type=jnp.float32)
    xcont = xc_ref[...]
    x_hi = xcont.astype(jnp.bfloat16)
    x_lo = (xcont - x_hi.astype(jnp.float32)).astype(jnp.bfloat16)
    cw = cw_ref[...]
    cw_hi = cw.astype(jnp.bfloat16)
    cw_lo = (cw - cw_hi.astype(jnp.float32)).astype(jnp.bfloat16)
    xc = (dot(x_hi, cw_hi) + dot(x_hi, cw_lo) + dot(x_lo, cw_hi))
    xc = xc + cb_ref[...]
    ex = ex_ref[...]
    h = dot(ex, w1e_ref[...])
    h = h + dot(xc, w1c_ref[...])
    h = jnp.maximum(h + b1_ref[...], 0.0)
    h2 = dot(h, w2_ref[...])
    h2 = jnp.maximum(h2 + b2_ref[...], 0.0)
    mlp = dot(h2, w3_ref[...])
    lin = (jnp.sum(fcv_ref[...], axis=1, keepdims=True)
           + jnp.sum(xcont, axis=1, keepdims=True))
    s = _split_dot(ex, se_ref[...]) + _split_dot(xc, sc_ref[...])
    q = _split_dot(ex * ex, se_ref[...]) + _split_dot(xc * xc, sc_ref[...])
    inter = 0.5 * jnp.sum(s * s - q, axis=1, keepdims=True)
    out_ref[...] = lin + inter + mlp + c0_ref[...]


def _tc_dense(ex_flat, x_continuous, fc_vals, cont_w, cont_b, w1e, w1c, b1,
              w2, b2, w3, s_e, s_c, c0):
    grid = (_B // _TB,)
    const = lambda i: (0, 0)
    return pl.pallas_call(
        _tc_body,
        grid=grid,
        in_specs=[
            pl.BlockSpec((_TB, _ED), lambda i: (i, 0)),
            pl.BlockSpec((_TB, _CP), lambda i: (i, 0)),
            pl.BlockSpec((_TB, _F), lambda i: (i, 0)),
            pl.BlockSpec((_CP, _CD), const),
            pl.BlockSpec((1, _CD), const),
            pl.BlockSpec((_ED, 256), const),
            pl.BlockSpec((_CD, 256), const),
            pl.BlockSpec((1, 256), const),
            pl.BlockSpec((256, 128), const),
            pl.BlockSpec((1, 128), const),
            pl.BlockSpec((128, 1), const),
            pl.BlockSpec((_ED, _D), const),
            pl.BlockSpec((_CD, _D), const),
            pl.BlockSpec((1, 1), const),
        ],
        out_specs=pl.BlockSpec((_TB, 1), lambda i: (i, 0)),
        out_shape=jax.ShapeDtypeStruct((_B, 1), jnp.float32),
    )(ex_flat, x_continuous, fc_vals, cont_w, cont_b, w1e, w1c, b1, w2, b2,
      w3, s_e, s_c, c0)


def kernel(x_categorical, x_continuous, emb_table, fc_table, bias, cont_w,
           cont_b, w1, b1, w2, b2, w3, b3):
    idx = x_categorical.astype(jnp.int32).reshape(_N)
    fc16 = fc_table.reshape(_V // 16, 16)
    rows, fcv = _sc_gather_fn()(emb_table, fc16, idx)
    ex_flat = rows.reshape(_B, _ED)
    fc_vals = fcv.reshape(_B, _F)

    w1e = w1[:_ED]
    w1c = w1[_ED:]
    s_e = jnp.tile(jnp.eye(_D, dtype=jnp.bfloat16), (_F, 1))
    s_c = jnp.tile(jnp.eye(_D, dtype=jnp.bfloat16), (_C, 1))
    c0 = (bias[0] + b3[0]).reshape(1, 1)

    xcont_p = jnp.pad(x_continuous, ((0, 0), (0, _CP - _C)))
    cont_w_p = jnp.pad(cont_w, ((0, _CP - _C), (0, 0)))
    y = _tc_dense(ex_flat, xcont_p, fc_vals, cont_w_p,
                  cont_b.reshape(1, _CD), w1e, w1c, b1.reshape(1, 256), w2,
                  b2.reshape(1, 128), w3, s_e, s_c, c0)
    return y.reshape(_B)


# speedup vs baseline: 2.0396x; 1.0533x over previous
import functools

import jax
import jax.numpy as jnp
from jax import lax
from jax.experimental import pallas as pl
from jax.experimental.pallas import tpu as pltpu
from jax.experimental.pallas import tpu_sc as plsc

_V = 1000000
_B = 16384
_F = 26
_D = 32
_C = 19
_CD = _C * _D
_ED = _F * _D
_NW = 32
_N = _B * _F
_G = 128
_NROWS = _N // _G
_ROWS_PER_W = _NROWS // _NW
_SUB = 8
_STEPS = _ROWS_PER_W // _SUB

_TB = 1024
_CP = 32


@functools.cache
def _sc_gather_fn():
    mesh = plsc.VectorSubcoreMesh(core_axis_name="c", subcore_axis_name="s")

    @functools.partial(
        pl.kernel,
        mesh=mesh,
        compiler_params=pltpu.CompilerParams(use_tc_tiling_on_sc=False,
                                             needs_layout_passes=False),
        out_type=[
            jax.ShapeDtypeStruct((_NROWS, _G, _D), jnp.float32),
            jax.ShapeDtypeStruct((_N,), jnp.float32),
        ],
        scratch_types=[
            pltpu.VMEM((_SUB * _G,), jnp.int32),
            pltpu.VMEM((_SUB * _G,), jnp.int32),
            pltpu.VMEM((_SUB, _G, _D), jnp.float32),
            pltpu.VMEM((_SUB * _G, 16), jnp.float32),
            pltpu.VMEM((_SUB * _G,), jnp.float32),
            pltpu.SemaphoreType.DMA,
            pltpu.SemaphoreType.DMA,
        ],
    )
    def sc_gather(emb_hbm, fc16_hbm, idx_hbm, out_e, out_f,
                  idx_v, ridx_v, rows_v, fcrows_v, fcv_v, sem_e, sem_f):
        wid = lax.axis_index("s") * 2 + lax.axis_index("c")
        base = wid * _ROWS_PER_W
        iota = lax.iota(jnp.int32, 16)
        ck = _SUB * _G

        def step(s, carry):
            row0 = base + s * _SUB
            k0 = row0 * _G
            pltpu.sync_copy(idx_hbm.at[pl.ds(k0, ck)], idx_v)
            emb_cp = [
                pltpu.async_copy(emb_hbm.at[idx_v.at[pl.ds(j * _G, _G)]],
                                 rows_v.at[j], sem_e)
                for j in range(_SUB)
            ]
            def rbody(t, c):
                v = idx_v[pl.ds(t * 16, 16)]
                ridx_v[pl.ds(t * 16, 16)] = jnp.right_shift(v, 4)
                return c

            lax.fori_loop(0, ck // 16, rbody, 0)
            fc_cp = [
                pltpu.async_copy(fc16_hbm.at[ridx_v.at[pl.ds(j * _G, _G)]],
                                 fcrows_v.at[pl.ds(j * _G, _G)], sem_f)
                for j in range(_SUB)
            ]
            for cp in emb_cp:
                cp.wait()
            for cp in fc_cp:
                cp.wait()

            def ebody(t, c):
                v = idx_v[pl.ds(t * 16, 16)]
                col = jnp.bitwise_and(v, 15)
                kvec = t * 16 + iota
                fcv_v[pl.ds(t * 16, 16)] = plsc.load_gather(
                    fcrows_v, [kvec, col])
                return c

            lax.fori_loop(0, ck // 16, ebody, 0)
            pltpu.sync_copy(rows_v, out_e.at[pl.ds(row0, _SUB)])
            pltpu.sync_copy(fcv_v, out_f.at[pl.ds(k0, ck)])
            return carry

        lax.fori_loop(0, _STEPS, step, 0)

    return sc_gather


def _tc_body(ex_ref, xc_ref, fcv_ref, cw_ref, cb_ref, w1e_ref, w1c_ref,
             b1_ref, w2_ref, b2_ref, w3_ref, se_ref, sc_ref, c0_ref,
             out_ref):
    dot = functools.partial(jnp.dot, preferred_element_type=jnp.float32)
    xcont = xc_ref[...]
    xc = dot(xcont, cw_ref[...]) + cb_ref[...]
    ex = ex_ref[...]
    h = dot(ex, w1e_ref[...])
    h = h + dot(xc, w1c_ref[...])
    h = jnp.maximum(h + b1_ref[...], 0.0)
    h2 = dot(h, w2_ref[...])
    h2 = jnp.maximum(h2 + b2_ref[...], 0.0)
    mlp = dot(h2, w3_ref[...])
    lin = (jnp.sum(fcv_ref[...], axis=1, keepdims=True)
           + jnp.sum(xcont, axis=1, keepdims=True))
    s = dot(ex, se_ref[...]) + dot(xc, sc_ref[...])
    q = dot(ex * ex, se_ref[...]) + dot(xc * xc, sc_ref[...])
    inter = 0.5 * jnp.sum(s * s - q, axis=1, keepdims=True)
    out_ref[...] = lin + inter + mlp + c0_ref[...]


def _tc_dense(ex_flat, x_continuous, fc_vals, cont_w, cont_b, w1e, w1c, b1,
              w2, b2, w3, s_e, s_c, c0):
    grid = (_B // _TB,)
    const = lambda i: (0, 0)
    return pl.pallas_call(
        _tc_body,
        grid=grid,
        in_specs=[
            pl.BlockSpec((_TB, _ED), lambda i: (i, 0)),
            pl.BlockSpec((_TB, _CP), lambda i: (i, 0)),
            pl.BlockSpec((_TB, _F), lambda i: (i, 0)),
            pl.BlockSpec((_CP, _CD), const),
            pl.BlockSpec((1, _CD), const),
            pl.BlockSpec((_ED, 256), const),
            pl.BlockSpec((_CD, 256), const),
            pl.BlockSpec((1, 256), const),
            pl.BlockSpec((256, 128), const),
            pl.BlockSpec((1, 128), const),
            pl.BlockSpec((128, 1), const),
            pl.BlockSpec((_ED, _D), const),
            pl.BlockSpec((_CD, _D), const),
            pl.BlockSpec((1, 1), const),
        ],
        out_specs=pl.BlockSpec((_TB, 1), lambda i: (i, 0)),
        out_shape=jax.ShapeDtypeStruct((_B, 1), jnp.float32),
    )(ex_flat, x_continuous, fc_vals, cont_w, cont_b, w1e, w1c, b1, w2, b2,
      w3, s_e, s_c, c0)


def kernel(x_categorical, x_continuous, emb_table, fc_table, bias, cont_w,
           cont_b, w1, b1, w2, b2, w3, b3):
    idx = x_categorical.astype(jnp.int32).reshape(_N)
    fc16 = fc_table.reshape(_V // 16, 16)
    rows, fcv = _sc_gather_fn()(emb_table, fc16, idx)
    ex_flat = rows.reshape(_B, _ED)
    fc_vals = fcv.reshape(_B, _F)

    w1e = w1[:_ED]
    w1c = w1[_ED:]
    s_e = jnp.tile(jnp.eye(_D, dtype=jnp.float32), (_F, 1))
    s_c = jnp.tile(jnp.eye(_D, dtype=jnp.float32), (_C, 1))
    c0 = (bias[0] + b3[0]).reshape(1, 1)

    xcont_p = jnp.pad(x_continuous, ((0, 0), (0, _CP - _C)))
    cont_w_p = jnp.pad(cont_w, ((0, _CP - _C), (0, 0)))
    y = _tc_dense(ex_flat, xcont_p, fc_vals, cont_w_p,
                  cont_b.reshape(1, _CD), w1e, w1c, b1.reshape(1, 256), w2,
                  b2.reshape(1, 128), w3, s_e, s_c, c0)
    return y.reshape(_B)


# speedup vs baseline: 2.0731x; 1.0164x over previous
import functools

import jax
import jax.numpy as jnp
from jax import lax
from jax.experimental import pallas as pl
from jax.experimental.pallas import tpu as pltpu
from jax.experimental.pallas import tpu_sc as plsc

_V = 1000000
_B = 16384
_F = 26
_D = 32
_C = 19
_CD = _C * _D
_ED = _F * _D
_NW = 32
_N = _B * _F
_G = 128
_SUB = 4

_TB = 1024
_CP = 32


@functools.cache
def _sc_gather_fn(nrows):
    rows_per_w = nrows // _NW
    steps = rows_per_w // _SUB
    mesh = plsc.VectorSubcoreMesh(core_axis_name="c", subcore_axis_name="s")

    @functools.partial(
        pl.kernel,
        mesh=mesh,
        compiler_params=pltpu.CompilerParams(use_tc_tiling_on_sc=False,
                                             needs_layout_passes=False),
        out_type=[
            jax.ShapeDtypeStruct((nrows, _G, _D), jnp.float32),
            jax.ShapeDtypeStruct((nrows * _G,), jnp.float32),
        ],
        scratch_types=[
            pltpu.VMEM((_SUB * _G,), jnp.int32),
            pltpu.VMEM((_SUB * _G,), jnp.int32),
            pltpu.VMEM((_SUB, _G, _D), jnp.float32),
            pltpu.VMEM((_SUB * _G, 16), jnp.float32),
            pltpu.VMEM((_SUB * _G,), jnp.float32),
            pltpu.SemaphoreType.DMA,
            pltpu.SemaphoreType.DMA,
        ],
    )
    def sc_gather(emb_hbm, fc16_hbm, idx_hbm, out_e, out_f,
                  idx_v, ridx_v, rows_v, fcrows_v, fcv_v, sem_e, sem_f):
        wid = lax.axis_index("s") * 2 + lax.axis_index("c")
        base = wid * rows_per_w
        iota = lax.iota(jnp.int32, 16)
        ck = _SUB * _G

        def step(s, carry):
            row0 = base + s * _SUB
            k0 = row0 * _G
            pltpu.sync_copy(idx_hbm.at[pl.ds(k0, ck)], idx_v)
            emb_cp = [
                pltpu.async_copy(emb_hbm.at[idx_v.at[pl.ds(j * _G, _G)]],
                                 rows_v.at[j], sem_e)
                for j in range(_SUB)
            ]
            def rbody(t, c):
                v = idx_v[pl.ds(t * 16, 16)]
                ridx_v[pl.ds(t * 16, 16)] = jnp.right_shift(v, 4)
                return c

            lax.fori_loop(0, ck // 16, rbody, 0)
            fc_cp = [
                pltpu.async_copy(fc16_hbm.at[ridx_v.at[pl.ds(j * _G, _G)]],
                                 fcrows_v.at[pl.ds(j * _G, _G)], sem_f)
                for j in range(_SUB)
            ]
            for cp in emb_cp:
                cp.wait()
            for cp in fc_cp:
                cp.wait()

            def ebody(t, c):
                v = idx_v[pl.ds(t * 16, 16)]
                col = jnp.bitwise_and(v, 15)
                kvec = t * 16 + iota
                fcv_v[pl.ds(t * 16, 16)] = plsc.load_gather(
                    fcrows_v, [kvec, col])
                return c

            lax.fori_loop(0, ck // 16, ebody, 0)
            pltpu.sync_copy(rows_v, out_e.at[pl.ds(row0, _SUB)])
            pltpu.sync_copy(fcv_v, out_f.at[pl.ds(k0, ck)])
            return carry

        lax.fori_loop(0, steps, step, 0)

    return sc_gather


def _tc_body(ex_ref, xc_ref, fcv_ref, cw_ref, cb_ref, w1e_ref, w1c_ref,
             b1_ref, w2_ref, b2_ref, w3_ref, se_ref, sc_ref, c0_ref,
             out_ref):
    dot = functools.partial(jnp.dot, preferred_element_type=jnp.float32)
    xcont = xc_ref[...]
    xc = dot(xcont, cw_ref[...]) + cb_ref[...]
    ex = ex_ref[...]
    h = dot(ex, w1e_ref[...])
    h = h + dot(xc, w1c_ref[...])
    h = jnp.maximum(h + b1_ref[...], 0.0)
    h2 = dot(h, w2_ref[...])
    h2 = jnp.maximum(h2 + b2_ref[...], 0.0)
    mlp = dot(h2, w3_ref[...])
    lin = (jnp.sum(fcv_ref[...], axis=1, keepdims=True)
           + jnp.sum(xcont, axis=1, keepdims=True))
    s = dot(ex, se_ref[...]) + dot(xc, sc_ref[...])
    q = dot(ex * ex, se_ref[...]) + dot(xc * xc, sc_ref[...])
    inter = 0.5 * jnp.sum(s * s - q, axis=1, keepdims=True)
    out_ref[...] = lin + inter + mlp + c0_ref[...]


def _tc_dense(ex_flat, x_continuous, fc_vals, cont_w, cont_b, w1e, w1c, b1,
              w2, b2, w3, s_e, s_c, c0):
    nb = ex_flat.shape[0]
    grid = (nb // _TB,)
    const = lambda i: (0, 0)
    return pl.pallas_call(
        _tc_body,
        grid=grid,
        in_specs=[
            pl.BlockSpec((_TB, _ED), lambda i: (i, 0)),
            pl.BlockSpec((_TB, _CP), lambda i: (i, 0)),
            pl.BlockSpec((_TB, _F), lambda i: (i, 0)),
            pl.BlockSpec((_CP, _CD), const),
            pl.BlockSpec((1, _CD), const),
            pl.BlockSpec((_ED, 256), const),
            pl.BlockSpec((_CD, 256), const),
            pl.BlockSpec((1, 256), const),
            pl.BlockSpec((256, 128), const),
            pl.BlockSpec((1, 128), const),
            pl.BlockSpec((128, 1), const),
            pl.BlockSpec((_ED, _D), const),
            pl.BlockSpec((_CD, _D), const),
            pl.BlockSpec((1, 1), const),
        ],
        out_specs=pl.BlockSpec((_TB, 1), lambda i: (i, 0)),
        out_shape=jax.ShapeDtypeStruct((nb, 1), jnp.float32),
    )(ex_flat, x_continuous, fc_vals, cont_w, cont_b, w1e, w1c, b1, w2, b2,
      w3, s_e, s_c, c0)


def kernel(x_categorical, x_continuous, emb_table, fc_table, bias, cont_w,
           cont_b, w1, b1, w2, b2, w3, b3):
    idx = x_categorical.astype(jnp.int32).reshape(_N)
    fc16 = fc_table.reshape(_V // 16, 16)

    w1e = w1[:_ED]
    w1c = w1[_ED:]
    s_e = jnp.tile(jnp.eye(_D, dtype=jnp.float32), (_F, 1))
    s_c = jnp.tile(jnp.eye(_D, dtype=jnp.float32), (_C, 1))
    c0 = (bias[0] + b3[0]).reshape(1, 1)
    xcont_p = jnp.pad(x_continuous, ((0, 0), (0, _CP - _C)))
    cont_w_p = jnp.pad(cont_w, ((0, _CP - _C), (0, 0)))

    hb = _B // 2
    hn = _N // 2
    sc = _sc_gather_fn(hn // _G)
    ys = []
    gathered = [sc(emb_table, fc16, lax.slice_in_dim(idx, i * hn, (i + 1) * hn))
                for i in range(2)]
    for i, (rows, fcv) in enumerate(gathered):
        ex_flat = rows.reshape(hb, _ED)
        fc_vals = fcv.reshape(hb, _F)
        y = _tc_dense(ex_flat,
                      lax.slice_in_dim(xcont_p, i * hb, (i + 1) * hb),
                      fc_vals, cont_w_p, cont_b.reshape(1, _CD), w1e, w1c,
                      b1.reshape(1, 256), w2, b2.reshape(1, 128), w3, s_e,
                      s_c, c0)
        ys.append(y.reshape(hb))
    return jnp.concatenate(ys)
